# baseline (speedup 1.0000x reference)
import functools

import numpy as np
import jax
import jax.numpy as jnp
from jax import lax
from jax.experimental import pallas as pl
from jax.experimental.pallas import tpu as pltpu
from jax.experimental.pallas import tpu_sc as plsc

_N, _C, _L, _T, _V = 32, 256, 6, 64, 25
_INTER = _C // 4
_K = 3
_EPS = 1e-5
_VP = 32
_P2 = _L * _VP
_GROUPS = [[20], [1, 2, 4, 8], [3, 5, 9, 0], [6, 10, 12, 16], [7, 11, 13, 17],
           [21, 22, 23, 24, 14, 18], [15, 19]]
_LAYERS = [_GROUPS[i] + _GROUPS[i + 1] for i in range(len(_GROUPS) - 1)]

_POOL_NP = np.zeros((_P2, _L), np.float32)
for _i, _idxs in enumerate(_LAYERS):
    for _j in _idxs:
        _POOL_NP[_i * _VP + _j, _i] = 1.0 / len(_idxs)
_VALID_NP = np.zeros((1, _P2), np.float32)
for _i in range(_L):
    _VALID_NP[0, _i * _VP:_i * _VP + _V] = 1.0

_HI = lax.Precision.HIGHEST
_DEF = lax.Precision.DEFAULT


def _maxt_body(x_ref, o_ref):
    xb = x_ref[0]
    cb = xb.shape[0]
    m = jnp.max(xb.reshape(cb, _L, 8, 8 * _V), axis=2)
    m = jnp.max(m.reshape(cb, _L, 8, _V), axis=2)
    pad = jnp.zeros((cb, _L, _VP - _V), jnp.float32)
    o_ref[...] = jnp.concatenate([m, pad], axis=-1)[None]


def _mid1_body(xt_ref, pool_ref, valid_ref, cw_ref, cb_ref, bn1w_ref,
               bn1b_ref, pdp_ref, xst_ref):
    pool = pool_ref[...]
    valid = valid_ref[...]
    cw = cw_ref[...]
    h = jnp.stack([jnp.dot(cw, xt_ref[i],
                           preferred_element_type=jnp.float32,
                           precision=_DEF) for i in range(_N)],
                  axis=0)
    h = h + cb_ref[...][None]
    cnt = float(_N * _L * _V)
    hm = h * valid[None]
    m1 = jnp.sum(jnp.sum(hm, axis=0), axis=-1, keepdims=True) / cnt
    d = (h - m1[None]) * valid[None]
    v1 = jnp.sum(jnp.sum(d * d, axis=0), axis=-1, keepdims=True) / cnt
    h = (h - m1[None]) / jnp.sqrt(v1[None] + _EPS)
    h = h * bn1w_ref[...][None] + bn1b_ref[...][None]
    h = jnp.maximum(h, 0.0)
    xs = jnp.stack([jnp.dot(h[i], pool,
                            preferred_element_type=jnp.float32,
                            precision=_HI) for i in range(_N)],
                   axis=0)
    xst = jnp.stack([xs[i].T for i in range(_N)], axis=0)
    inner = jnp.stack([jnp.dot(xst[i], xs[i],
                               preferred_element_type=jnp.float32,
                               precision=_DEF) for i in range(_N)],
                      axis=0)
    xx = jnp.sum(xs * xs, axis=1)
    pd = -xx[:, None, :] + 2.0 * inner - xx[:, :, None]
    pad = jnp.full((_N, _L, 16 - _L), -3.4e38, jnp.float32)
    pdp_ref[...] = jnp.concatenate([pd, pad], axis=-1).reshape(_N * _L, 16)
    xst_ref[...] = xst.reshape(_N * _L, _INTER)


def _topk_sc_call(pdp):
    info = plsc.get_sparse_core_info()
    nw = info.num_cores * info.num_subcores
    rows = _N * _L
    rpw = rows // nw
    mesh = plsc.VectorSubcoreMesh(core_axis_name="c", subcore_axis_name="s")

    @functools.partial(
        pl.kernel, mesh=mesh,
        out_type=jax.ShapeDtypeStruct((nw, rpw * _K, 16), jnp.float32),
        scratch_types=[
            pltpu.VMEM((rpw, 16), jnp.float32),
            pltpu.VMEM((rpw * _K, 16), jnp.float32),
        ],
    )
    def _topk(pd_hbm, out_hbm, pdv, ohv):
        wid = lax.axis_index("s") * info.num_cores + lax.axis_index("c")
        pltpu.sync_copy(pd_hbm.at[wid], pdv)
        ids = lax.iota(jnp.int32, 16)
        for r in range(rpw):
            row = pdv[r, :]
            for k in range(_K):
                m = row
                for s in (1, 2, 4, 8):
                    m = jnp.maximum(m, jnp.take(m, ids ^ s))
                cand = jnp.where(row == m, ids, 127)
                for s in (1, 2, 4, 8):
                    cand = jnp.minimum(cand, jnp.take(cand, ids ^ s))
                oh = ids == cand
                ohv[r * _K + k, :] = jnp.where(oh, 1.0, 0.0)
                row = jnp.where(oh, -3.4e38, row)
        pltpu.sync_copy(ohv, out_hbm.at[wid])

    return _topk(pdp.reshape(nw, rpw, 16)).reshape(rows, _K, 16)


def _mid2_body(xst_ref, oh_ref, ewt_ref, bn2w_ref, bn2b_ref, awt_ref,
               ab_ref, gate_ref):
    xst = xst_ref[...].reshape(_N, _L, _INTER)
    oh = oh_ref[...].reshape(_N, _L, _K, 16)
    feats = []
    for k in range(_K):
        onehot = oh[:, :, k, :_L]
        feats.append(jnp.sum(onehot[:, :, :, None] * xst[:, None, :, :],
                             axis=2))
    feat = jnp.stack(feats, axis=2)
    xrep = jnp.broadcast_to(xst[:, :, None, :], feat.shape)
    feature = jnp.concatenate([feat - xrep, xrep], axis=3)
    f2 = feature.reshape(_N * _L * _K, 2 * _INTER)
    e = jnp.dot(f2, ewt_ref[...], preferred_element_type=jnp.float32,
                precision=_HI)
    m2 = jnp.mean(e, axis=0, keepdims=True)
    v2 = jnp.mean((e - m2) ** 2, axis=0, keepdims=True)
    e = (e - m2) / jnp.sqrt(v2 + _EPS) * bn2w_ref[...] + bn2b_ref[...]
    e = jnp.where(e >= 0, e, 0.2 * e)
    att = jnp.max(e.reshape(_N, _L, _K, _INTER), axis=2)
    att = jnp.dot(att.reshape(_N * _L, _INTER), awt_ref[...],
                  preferred_element_type=jnp.float32,
                  precision=_HI) + ab_ref[...]
    gate_ref[...] = jax.nn.sigmoid(att)


def _gate_body(x_ref, g_ref, o_ref):
    li = pl.program_id(2)
    xb = x_ref[0]
    g = g_ref[0]
    rowmask = lax.broadcasted_iota(jnp.int32, (_L, 1), 0) == li
    gl = jnp.sum(jnp.where(rowmask, g, 0.0), axis=0)
    contrib = xb * gl[:, None, None]

    @pl.when(li == 0)
    def _():
        o_ref[...] = contrib[None]

    @pl.when(li > 0)
    def _():
        o_ref[...] += contrib[None]


def kernel(x, conv_down_w, conv_down_b, bn1_w, bn1_b, edge_w, bn2_w, bn2_b,
           agg_w, agg_b):
    n, c, l, t, v = x.shape
    pool = jnp.asarray(_POOL_NP)
    valid = jnp.asarray(_VALID_NP)

    x48 = x.reshape(n, c, _L * 8, 8 * _V)

    CBA = 128
    xt = pl.pallas_call(
        _maxt_body,
        grid=(n, c // CBA),
        in_specs=[pl.BlockSpec((1, CBA, _L * 8, 8 * _V),
                               lambda i, j: (i, j, 0, 0))],
        out_specs=pl.BlockSpec((1, CBA, _L, _VP), lambda i, j: (i, j, 0, 0)),
        out_shape=jax.ShapeDtypeStruct((n, c, _L, _VP), jnp.float32),
    )(x48)

    xt3 = xt.reshape(n, c, _P2)
    pdp, xst = pl.pallas_call(
        _mid1_body,
        out_shape=[jax.ShapeDtypeStruct((n * _L, 16), jnp.float32),
                   jax.ShapeDtypeStruct((n * _L, _INTER), jnp.float32)],
    )(xt3, pool, valid, conv_down_w, conv_down_b.reshape(-1, 1),
      bn1_w.reshape(-1, 1), bn1_b.reshape(-1, 1))

    oh = _topk_sc_call(pdp).reshape(n * _L * _K, 16)

    gate = pl.pallas_call(
        _mid2_body,
        out_shape=jax.ShapeDtypeStruct((n * _L, c), jnp.float32),
    )(xst, oh, edge_w.T, bn2_w.reshape(1, -1), bn2_b.reshape(1, -1),
      agg_w.T, agg_b.reshape(1, -1))
    gate3 = gate.reshape(n, _L, c)

    CBC = 128
    out = pl.pallas_call(
        _gate_body,
        grid=(n, c // CBC, _L),
        in_specs=[
            pl.BlockSpec((1, CBC, 8, 8 * _V),
                         lambda i, j, li: (i, j, li, 0)),
            pl.BlockSpec((1, _L, CBC), lambda i, j, li: (i, 0, j)),
        ],
        out_specs=pl.BlockSpec((1, CBC, 8, 8 * _V),
                               lambda i, j, li: (i, j, 0, 0)),
        out_shape=jax.ShapeDtypeStruct((n, c, 8, 8 * _V), jnp.float32),
    )(x48, gate3)
    return out.reshape(n, c, _T, _V)
